# Initial kernel scaffold; baseline (speedup 1.0000x reference)
import jax
import jax.numpy as jnp
from jax.experimental import pallas as pl


def kernel(input, adj, gn_func, nn_func, weight, bias):
    raise NotImplementedError("write your pallas kernel here")


# speedup vs baseline: 1.0371x; 1.0371x over previous
import jax
import jax.numpy as jnp
from jax.experimental import pallas as pl
from jax.experimental.pallas import tpu as pltpu


def _gcn_kernel(adj_ref, x_ref, w_ref, b_ref, out_ref, support_ref):
    i = pl.program_id(0)

    @pl.when(i == 0)
    def _():
        support_ref[...] = jnp.maximum(
            jnp.dot(x_ref[...], w_ref[...], preferred_element_type=jnp.float32), 0.0
        )

    out_ref[...] = (
        jnp.dot(adj_ref[...], support_ref[...], preferred_element_type=jnp.float32)
        + b_ref[...]
    )


def kernel(input, adj, gn_func, nn_func, weight, bias):
    x = input
    n, d_in = x.shape
    d_out = weight.shape[1]
    bi = 200 if n % 200 == 0 else n
    ni = n // bi
    b2 = bias.reshape(1, d_out).astype(jnp.float32)

    out = pl.pallas_call(
        _gcn_kernel,
        grid=(ni,),
        in_specs=[
            pl.BlockSpec((bi, n), lambda i: (i, 0)),
            pl.BlockSpec((n, d_in), lambda i: (0, 0)),
            pl.BlockSpec((d_in, d_out), lambda i: (0, 0)),
            pl.BlockSpec((1, d_out), lambda i: (0, 0)),
        ],
        out_specs=pl.BlockSpec((bi, d_out), lambda i: (i, 0)),
        out_shape=jax.ShapeDtypeStruct((n, d_out), jnp.float32),
        scratch_shapes=[pltpu.VMEM((n, d_out), jnp.float32)],
    )(adj, x, weight, b2)
    return out


# speedup vs baseline: 1.0398x; 1.0026x over previous
import jax
import jax.numpy as jnp
from jax.experimental import pallas as pl
from jax.experimental.pallas import tpu as pltpu


def _gcn_kernel(adj_ref, x_ref, w_ref, b_ref, out_ref, support_ref):
    i = pl.program_id(0)

    @pl.when(i == 0)
    def _():
        support_ref[...] = jnp.maximum(
            jnp.dot(x_ref[...], w_ref[...], preferred_element_type=jnp.float32), 0.0
        )

    out_ref[...] = (
        jnp.dot(adj_ref[...], support_ref[...], preferred_element_type=jnp.float32)
        + b_ref[...]
    )


def kernel(input, adj, gn_func, nn_func, weight, bias):
    x = input
    n, d_in = x.shape
    d_out = weight.shape[1]
    bi = 400 if n % 400 == 0 else n
    ni = n // bi
    b2 = bias.reshape(1, d_out).astype(jnp.float32)

    out = pl.pallas_call(
        _gcn_kernel,
        grid=(ni,),
        in_specs=[
            pl.BlockSpec((bi, n), lambda i: (i, 0)),
            pl.BlockSpec((n, d_in), lambda i: (0, 0)),
            pl.BlockSpec((d_in, d_out), lambda i: (0, 0)),
            pl.BlockSpec((1, d_out), lambda i: (0, 0)),
        ],
        out_specs=pl.BlockSpec((bi, d_out), lambda i: (i, 0)),
        out_shape=jax.ShapeDtypeStruct((n, d_out), jnp.float32),
        scratch_shapes=[pltpu.VMEM((n, d_out), jnp.float32)],
    )(adj, x, weight, b2)
    return out
